# baseline (speedup 1.0000x reference)
import jax
import jax.numpy as jnp
from jax.experimental import pallas as pl


def _apply_mpl(x, ei, p):
    src, dst = ei[0], ei[1]
    m = jax.nn.leaky_relu(jnp.concatenate([x[src], x[dst]], axis=-1) @ p["msg"]["W"] + p["msg"]["b"])
    agg = jax.ops.segment_sum(m, dst, num_segments=x.shape[0])
    h = jax.nn.leaky_relu(jnp.concatenate([x, agg], axis=-1) @ p["node"]["W"] + p["node"]["b"])
    return h + x @ p["skip"]["W"] + p["skip"]["b"]


def _apply_bn(x, p):
    mu = x.mean(axis=0)
    var = x.var(axis=0)
    return (x - mu) / jnp.sqrt(var + 1e-5) * p["gamma"] + p["beta"]


def _head_body(x_ref, w1_ref, b1_ref, w2_ref, b2_ref, g_ref, be_ref, o_ref):
    x = x_ref[...]
    h = jax.nn.leaky_relu(x @ w1_ref[...] + b1_ref[...])
    y = h @ w2_ref[...] + b2_ref[...]
    mu = y.mean(axis=-1, keepdims=True)
    var = ((y - mu) ** 2).mean(axis=-1, keepdims=True)
    o_ref[...] = (y - mu) / jnp.sqrt(var + 1e-5) * g_ref[...] + be_ref[...]


def kernel(z, e_bot, e_mid, e_top, up1, up2, params):
    zz = z @ params["up_mlp"]["W"] + params["up_mlp"]["b"]
    zz = jnp.transpose(zz, (0, 2, 1))
    zz = jax.nn.leaky_relu(zz @ params["lup1"]["W"] + params["lup1"]["b"])
    zz = zz @ params["lup2"]["W"] + params["lup2"]["b"]
    x = zz[0]
    x = _apply_mpl(x, e_bot, params["mpl_bottom"])
    skip = _apply_mpl(x[up1], e_mid, params["l0_skip"])
    h = _apply_mpl(x, e_bot, params["l0_mpl1"])
    h = _apply_mpl(h[up1], e_mid, params["l0_mpl2"])
    x = jax.nn.leaky_relu(_apply_bn(h + skip, params["l0_bn"]))
    skip = _apply_mpl(x[up2], e_top, params["l1_skip"])
    h = _apply_mpl(x, e_mid, params["l1_mpl1"])
    h = _apply_mpl(h[up2], e_top, params["l1_mpl2"])
    x = jax.nn.leaky_relu(_apply_bn(h + skip, params["l1_bn"]))
    x = _apply_mpl(x, e_top, params["final"])

    N = x.shape[0]
    BLK = 2000
    grid = (N + BLK - 1) // BLK
    out = pl.pallas_call(
        _head_body,
        grid=(grid,),
        in_specs=[
            pl.BlockSpec((BLK, 64), lambda i: (i, 0)),
            pl.BlockSpec((64, 32), lambda i: (0, 0)),
            pl.BlockSpec((32,), lambda i: (0,)),
            pl.BlockSpec((32, 3), lambda i: (0, 0)),
            pl.BlockSpec((3,), lambda i: (0,)),
            pl.BlockSpec((3,), lambda i: (0,)),
            pl.BlockSpec((3,), lambda i: (0,)),
        ],
        out_specs=pl.BlockSpec((BLK, 3), lambda i: (i, 0)),
        out_shape=jax.ShapeDtypeStruct((N, 3), jnp.float32),
    )(x, params["dec1"]["W"], params["dec1"]["b"], params["dec2"]["W"],
      params["dec2"]["b"], params["ln"]["gamma"], params["ln"]["beta"])
    return out


# speedup vs baseline: 2.7735x; 2.7735x over previous
import functools

import jax
import jax.numpy as jnp
from jax import lax
from jax.experimental import pallas as pl
from jax.experimental.pallas import tpu as pltpu
from jax.experimental.pallas import tpu_sc as plsc

LEAK = 0.01


def _leaky(x):
    return jnp.maximum(x, LEAK * x)


def _round_up(v, m):
    return (v + m - 1) // m * m


def _npad(n):
    return _round_up(n + 1, 128)


def _tc_head(z, params):
    upWT = params["up_mlp"]["W"].T
    upb = params["up_mlp"]["b"][:, None]
    zc = z.reshape(-1, 1)
    w1, b1 = params["lup1"]["W"], params["lup1"]["b"]
    w2, b2 = params["lup2"]["W"], params["lup2"]["b"]
    n, lat = upWT.shape

    def body(zc_ref, wt_ref, ub_ref, w1_ref, b1_ref, w2_ref, b2_ref, o_ref):
        v = jnp.dot(wt_ref[...], zc_ref[...],
                    preferred_element_type=jnp.float32) + ub_ref[...]
        t = _leaky(v * w1_ref[...] + b1_ref[...])
        o_ref[...] = jnp.dot(t, w2_ref[...],
                             preferred_element_type=jnp.float32) + b2_ref[...]

    return pl.pallas_call(
        body,
        out_shape=jax.ShapeDtypeStruct((n, lat), jnp.float32),
    )(zc, upWT, upb, w1, b1, w2, b2)


def _tc_pre(x, wsrc, wdst, bmsg, n_pad):
    n, din = x.shape
    d = wsrc.shape[1]
    d2 = d // 2
    blk = min(512, _round_up(n, 8))
    grid = (pl.cdiv(n, blk),)

    def body(x_ref, ws_ref, wd_ref, b_ref, ta_ref, tb_ref):
        xb = x_ref[...]
        a = jnp.dot(xb, ws_ref[...], preferred_element_type=jnp.float32)
        b = jnp.dot(xb, wd_ref[...], preferred_element_type=jnp.float32) + b_ref[...]
        ta_ref[0] = a[:, :d2]
        ta_ref[1] = a[:, d2:]
        tb_ref[0] = b[:, :d2]
        tb_ref[1] = b[:, d2:]

    ta, tb = pl.pallas_call(
        body,
        grid=grid,
        in_specs=[
            pl.BlockSpec((blk, din), lambda i: (i, 0)),
            pl.BlockSpec((din, d), lambda i: (0, 0)),
            pl.BlockSpec((din, d), lambda i: (0, 0)),
            pl.BlockSpec((d,), lambda i: (0,)),
        ],
        out_specs=[
            pl.BlockSpec((2, blk, d2), lambda i: (0, i, 0)),
            pl.BlockSpec((2, blk, d2), lambda i: (0, i, 0)),
        ],
        out_shape=[jax.ShapeDtypeStruct((2, n_pad, d2), jnp.float32)] * 2,
    )(x, wsrc, wdst, bmsg)
    return ta.reshape(2 * n_pad, d2), tb.reshape(2 * n_pad, d2)


def _tc_post(x, agg, wnx, wna0, wna1, bnode, wsk, bsk):
    n, din = x.shape
    dout = wnx.shape[1]
    d2 = agg.shape[2]
    blk = min(512, _round_up(n, 8))
    grid = (pl.cdiv(n, blk),)

    def body(x_ref, a0_ref, a1_ref, wnx_ref, wa0_ref, wa1_ref, bn_ref,
             wsk_ref, bsk_ref, o_ref):
        xb = x_ref[...]
        h = jnp.dot(xb, wnx_ref[...], preferred_element_type=jnp.float32)
        h = h + jnp.dot(a0_ref[0], wa0_ref[...], preferred_element_type=jnp.float32)
        h = h + jnp.dot(a1_ref[0], wa1_ref[...], preferred_element_type=jnp.float32)
        h = _leaky(h + bn_ref[...])
        o_ref[...] = h + jnp.dot(xb, wsk_ref[...],
                                 preferred_element_type=jnp.float32) + bsk_ref[...]

    return pl.pallas_call(
        body,
        grid=grid,
        in_specs=[
            pl.BlockSpec((blk, din), lambda i: (i, 0)),
            pl.BlockSpec((1, blk, d2), lambda i: (0, i, 0)),
            pl.BlockSpec((1, blk, d2), lambda i: (1, i, 0)),
            pl.BlockSpec((din, dout), lambda i: (0, 0)),
            pl.BlockSpec((d2, dout), lambda i: (0, 0)),
            pl.BlockSpec((d2, dout), lambda i: (0, 0)),
            pl.BlockSpec((dout,), lambda i: (0,)),
            pl.BlockSpec((din, dout), lambda i: (0, 0)),
            pl.BlockSpec((dout,), lambda i: (0,)),
        ],
        out_specs=pl.BlockSpec((blk, dout), lambda i: (i, 0)),
        out_shape=jax.ShapeDtypeStruct((n, dout), jnp.float32),
    )(x, agg, agg, wnx, wna0, wna1, bnode, wsk, bsk)


def _tc_bn(h, skip, gamma, beta):
    n, d = h.shape
    blk = 2560
    grid = (pl.cdiv(n, blk),)

    def sums_body(h_ref, s_ref, o_ref):
        y = h_ref[...] + s_ref[...]
        row = lax.broadcasted_iota(jnp.int32, (blk, 1), 0) + pl.program_id(0) * blk
        y = jnp.where(row < n, y, 0.0)

        @pl.when(pl.program_id(0) == 0)
        def _():
            o_ref[...] = jnp.zeros_like(o_ref)

        o_ref[0, :] += jnp.sum(y, axis=0)
        o_ref[1, :] += jnp.sum(y * y, axis=0)

    sums = pl.pallas_call(
        sums_body,
        grid=grid,
        in_specs=[pl.BlockSpec((blk, d), lambda i: (i, 0)),
                  pl.BlockSpec((blk, d), lambda i: (i, 0))],
        out_specs=pl.BlockSpec((8, d), lambda i: (0, 0)),
        out_shape=jax.ShapeDtypeStruct((8, d), jnp.float32),
    )(h, skip)

    def norm_body(h_ref, s_ref, m_ref, g_ref, b_ref, o_ref):
        y = h_ref[...] + s_ref[...]
        mu = m_ref[0, :] * (1.0 / n)
        var = m_ref[1, :] * (1.0 / n) - mu * mu
        o_ref[...] = _leaky((y - mu) * lax.rsqrt(var + 1e-5) * g_ref[...] + b_ref[...])

    return pl.pallas_call(
        norm_body,
        grid=grid,
        in_specs=[pl.BlockSpec((blk, d), lambda i: (i, 0)),
                  pl.BlockSpec((blk, d), lambda i: (i, 0)),
                  pl.BlockSpec((8, d), lambda i: (0, 0)),
                  pl.BlockSpec((d,), lambda i: (0,)),
                  pl.BlockSpec((d,), lambda i: (0,))],
        out_specs=pl.BlockSpec((blk, d), lambda i: (i, 0)),
        out_shape=jax.ShapeDtypeStruct((n, d), jnp.float32),
    )(h, skip, sums, gamma, beta)


def _tc_dec(x, params):
    n = x.shape[0]
    blk = 2000
    grid = (pl.cdiv(n, blk),)

    def body(x_ref, w1_ref, b1_ref, w2_ref, b2_ref, g_ref, be_ref, o_ref):
        h = _leaky(jnp.dot(x_ref[...], w1_ref[...],
                           preferred_element_type=jnp.float32) + b1_ref[...])
        y = jnp.dot(h, w2_ref[...], preferred_element_type=jnp.float32) + b2_ref[...]
        mu = y.mean(axis=-1, keepdims=True)
        var = ((y - mu) ** 2).mean(axis=-1, keepdims=True)
        o_ref[...] = (y - mu) * lax.rsqrt(var + 1e-5) * g_ref[...] + be_ref[...]

    return pl.pallas_call(
        body,
        grid=grid,
        in_specs=[
            pl.BlockSpec((blk, 64), lambda i: (i, 0)),
            pl.BlockSpec((64, 32), lambda i: (0, 0)),
            pl.BlockSpec((32,), lambda i: (0,)),
            pl.BlockSpec((32, 3), lambda i: (0, 0)),
            pl.BlockSpec((3,), lambda i: (0,)),
            pl.BlockSpec((3,), lambda i: (0,)),
            pl.BlockSpec((3,), lambda i: (0,)),
        ],
        out_specs=pl.BlockSpec((blk, 3), lambda i: (i, 0)),
        out_shape=jax.ShapeDtypeStruct((n, 3), jnp.float32),
    )(x, params["dec1"]["W"], params["dec1"]["b"], params["dec2"]["W"],
      params["dec2"]["b"], params["ln"]["gamma"], params["ln"]["beta"])


def _chunk_c(n_pad, d2):
    budget = 8 * 1024 * 1024 - n_pad * d2 * 4 - 384 * 1024
    per_tile = budget // 16 - 4096
    c = per_tile // (8 * d2 + 16)
    return max(128, min(1024, c // 128 * 128))


def _edge_epad(e):
    return 2048 * ((e + 2047) // 2048)


@functools.lru_cache(maxsize=None)
def _edge_kernel(n_pad, d2, e_pad):
    c = _chunk_c(n_pad, d2)
    t_per = e_pad // 16
    n_full = t_per // c
    rem = t_per - n_full * c
    k = c // 128
    k_rem = rem // 128
    z_rows = n_pad // 16
    nvec = d2 // 16

    mesh = plsc.VectorSubcoreMesh(core_axis_name="c", subcore_axis_name="s")

    def body(ta, tb, esrc0, esrc1, edst0, edst1, out, acc, idx_a, idx_b,
             idx_d2, buf_a, buf_b, sem):
        cid = lax.axis_index("c")
        sid = lax.axis_index("s")
        tile_base = sid * t_per
        zbase = sid * z_rows

        @pl.loop(0, c, unroll=8)
        def _zfill(r):
            for q in range(nvec):
                buf_a[r, pl.ds(q * 16, 16)] = jnp.zeros((16,), jnp.float32)

        off = 0
        while off < z_rows:
            sz = min(c, z_rows - off)
            pltpu.sync_copy(buf_a.at[pl.ds(0, sz)], acc.at[pl.ds(zbase + off, sz)])
            off += sz
        plsc.subcore_barrier()

        def do_chunk(base, csz, kk):
            @pl.when(cid == 0)
            def _():
                pltpu.sync_copy(esrc0.at[pl.ds(base, csz)], idx_a.at[pl.ds(0, csz)])
                pltpu.sync_copy(edst0.at[pl.ds(base, csz)], idx_b.at[pl.ds(0, csz)])

            @pl.when(cid == 1)
            def _():
                pltpu.sync_copy(esrc1.at[pl.ds(base, csz)], idx_a.at[pl.ds(0, csz)])
                pltpu.sync_copy(edst1.at[pl.ds(base, csz)], idx_b.at[pl.ds(0, csz)])

            for j in range(kk):
                pltpu.sync_copy(edst0.at[pl.ds(base + j * 128, 128)], idx_d2.at[j])
            descs = []
            for j in range(kk):
                descs.append(pltpu.async_copy(
                    ta.at[idx_a.at[pl.ds(j * 128, 128)]],
                    buf_a.at[pl.ds(j * 128, 128)], sem))
                descs.append(pltpu.async_copy(
                    tb.at[idx_b.at[pl.ds(j * 128, 128)]],
                    buf_b.at[pl.ds(j * 128, 128)], sem))
            for dsc in descs:
                dsc.wait()

            @pl.loop(0, csz, unroll=4)
            def _comp(r):
                for q in range(nvec):
                    a = buf_a[r, pl.ds(q * 16, 16)]
                    b = buf_b[r, pl.ds(q * 16, 16)]
                    v = a + b
                    buf_a[r, pl.ds(q * 16, 16)] = jnp.maximum(v, LEAK * v)

            for j in range(kk):
                pltpu.sync_copy(buf_a.at[pl.ds(j * 128, 128)],
                                acc.at[idx_d2.at[j]], add=True)

        @pl.loop(0, n_full)
        def _chunks(i):
            do_chunk(tile_base + i * c, c, k)

        if rem:
            do_chunk(tile_base + n_full * c, rem, k_rem)

        plsc.subcore_barrier()
        off = 0
        while off < z_rows:
            sz = min(c, z_rows - off)
            pltpu.sync_copy(acc.at[pl.ds(zbase + off, sz)],
                            out.at[cid, pl.ds(zbase + off, sz)])
            off += sz

    return pl.kernel(
        body,
        out_type=jax.ShapeDtypeStruct((2, n_pad, d2), jnp.float32),
        mesh=mesh,
        compiler_params=pltpu.CompilerParams(use_tc_tiling_on_sc=False),
        scratch_types=[
            pltpu.VMEM_SHARED((n_pad, d2), jnp.float32),
            pltpu.VMEM((c,), jnp.int32),
            pltpu.VMEM((c,), jnp.int32),
            pltpu.VMEM((k, 128), jnp.int32),
            pltpu.VMEM((c, d2), jnp.float32),
            pltpu.VMEM((c, d2), jnp.float32),
            pltpu.SemaphoreType.DMA,
        ],
    )


@functools.lru_cache(maxsize=None)
def _unpool_kernel(n_in, n_out_pad, d):
    rows = n_out_pad // 32
    cmax = max(128, min(1024, (256 * 1024) // (d * 4) // 8 * 8))
    mesh = plsc.VectorSubcoreMesh(core_axis_name="c", subcore_axis_name="s")

    def body(x_hbm, up_hbm, out, idx_v, buf, sem):
        cid = lax.axis_index("c")
        sid = lax.axis_index("s")
        wid = sid * 2 + cid
        base = wid * rows
        off = 0
        while off < rows:
            sz = min(cmax, rows - off)
            pltpu.sync_copy(up_hbm.at[pl.ds(base + off, sz)], idx_v.at[pl.ds(0, sz)])
            descs = []
            goff = 0
            while goff < sz:
                g = min(128, sz - goff)
                descs.append(pltpu.async_copy(
                    x_hbm.at[idx_v.at[pl.ds(goff, g)]],
                    buf.at[pl.ds(goff, g)], sem))
                goff += g
            for dsc in descs:
                dsc.wait()
            pltpu.sync_copy(buf.at[pl.ds(0, sz)], out.at[pl.ds(base + off, sz)])
            off += sz

    return pl.kernel(
        body,
        out_type=jax.ShapeDtypeStruct((n_out_pad, d), jnp.float32),
        mesh=mesh,
        compiler_params=pltpu.CompilerParams(use_tc_tiling_on_sc=False),
        scratch_types=[
            pltpu.VMEM((cmax,), jnp.int32),
            pltpu.VMEM((cmax, d), jnp.float32),
            pltpu.SemaphoreType.DMA,
        ],
    )


def _unpool(x, up, n_out):
    n_in, d = x.shape
    n_out_pad = _round_up(n_out, 256)
    up_pad = jnp.concatenate([up, jnp.zeros((n_out_pad - n_out,), jnp.int32)])
    out = _unpool_kernel(n_in, n_out_pad, d)(x, up_pad)
    return out[:n_out]


def _pad_edges(e, n, n_pad):
    e_cnt = e.shape[1]
    e_pad = _edge_epad(e_cnt)
    fill = e_pad - e_cnt
    esrc = jnp.concatenate([e[0], jnp.zeros((fill,), jnp.int32)])
    edst = jnp.concatenate([e[1], jnp.full((fill,), n, jnp.int32)])
    return esrc, esrc + n_pad, edst, edst + n_pad, e_pad


def _mpl(x, edges, mp):
    esrc0, esrc1, edst0, edst1, e_pad = edges
    n, din = x.shape
    n_pad = _npad(n)
    d = mp["msg"]["W"].shape[1]
    d2 = d // 2
    ta, tb = _tc_pre(x, mp["msg"]["W"][:din], mp["msg"]["W"][din:],
                     mp["msg"]["b"], n_pad)
    agg = _edge_kernel(n_pad, d2, e_pad)(ta, tb, esrc0, esrc1, edst0, edst1)
    wn = mp["node"]["W"]
    return _tc_post(x, agg, wn[:din], wn[din:din + d2], wn[din + d2:],
                    mp["node"]["b"], mp["skip"]["W"], mp["skip"]["b"])


N_BOT, N_MID, N_TOP = 3125, 12500, 50000


def kernel(z, e_bot, e_mid, e_top, up1, up2, params):
    p = params
    eb = _pad_edges(e_bot, N_BOT, _npad(N_BOT))
    em_mid = _pad_edges(e_mid, N_MID, _npad(N_MID))
    et_top = _pad_edges(e_top, N_TOP, _npad(N_TOP))

    x0 = _tc_head(z, p)
    x1 = _mpl(x0, eb, p["mpl_bottom"])

    x1u = _unpool(x1, up1, N_MID)
    skip = _mpl(x1u, em_mid, p["l0_skip"])
    h = _mpl(x1, eb, p["l0_mpl1"])
    hu = _unpool(h, up1, N_MID)
    h = _mpl(hu, em_mid, p["l0_mpl2"])
    x2 = _tc_bn(h, skip, p["l0_bn"]["gamma"], p["l0_bn"]["beta"])

    x2u = _unpool(x2, up2, N_TOP)
    skip = _mpl(x2u, et_top, p["l1_skip"])
    h = _mpl(x2, em_mid, p["l1_mpl1"])
    hu = _unpool(h, up2, N_TOP)
    h = _mpl(hu, et_top, p["l1_mpl2"])
    x3 = _tc_bn(h, skip, p["l1_bn"]["gamma"], p["l1_bn"]["beta"])

    x4 = _mpl(x3, et_top, p["final"])
    return _tc_dec(x4, p)


# speedup vs baseline: 3.3581x; 1.2108x over previous
import functools

import jax
import jax.numpy as jnp
from jax import lax
from jax.experimental import pallas as pl
from jax.experimental.pallas import tpu as pltpu
from jax.experimental.pallas import tpu_sc as plsc

LEAK = 0.01


def _leaky(x):
    return jnp.maximum(x, LEAK * x)


def _round_up(v, m):
    return (v + m - 1) // m * m


def _npad(n):
    return _round_up(n + 1, 128)


def _tc_head(z, params):
    upWT = params["up_mlp"]["W"].T
    upb = params["up_mlp"]["b"][:, None]
    zc = z.reshape(-1, 1)
    w1, b1 = params["lup1"]["W"], params["lup1"]["b"]
    w2, b2 = params["lup2"]["W"], params["lup2"]["b"]
    n, lat = upWT.shape

    def body(zc_ref, wt_ref, ub_ref, w1_ref, b1_ref, w2_ref, b2_ref, o_ref):
        v = jnp.dot(wt_ref[...], zc_ref[...],
                    preferred_element_type=jnp.float32) + ub_ref[...]
        t = _leaky(v * w1_ref[...] + b1_ref[...])
        o_ref[...] = jnp.dot(t, w2_ref[...],
                             preferred_element_type=jnp.float32) + b2_ref[...]

    return pl.pallas_call(
        body,
        out_shape=jax.ShapeDtypeStruct((n, lat), jnp.float32),
    )(zc, upWT, upb, w1, b1, w2, b2)


def _tc_pre(x, wsrc, wdst, bmsg, n_pad):
    n, din = x.shape
    d = wsrc.shape[1]
    d2 = d // 2
    blk = min(512, _round_up(n, 8))
    grid = (pl.cdiv(n, blk),)

    def body(x_ref, ws_ref, wd_ref, b_ref, ta_ref, tb_ref):
        xb = x_ref[...]
        a = jnp.dot(xb, ws_ref[...], preferred_element_type=jnp.float32)
        b = jnp.dot(xb, wd_ref[...], preferred_element_type=jnp.float32) + b_ref[...]
        ta_ref[0] = a[:, :d2]
        ta_ref[1] = a[:, d2:]
        tb_ref[0] = b[:, :d2]
        tb_ref[1] = b[:, d2:]

    ta, tb = pl.pallas_call(
        body,
        grid=grid,
        in_specs=[
            pl.BlockSpec((blk, din), lambda i: (i, 0)),
            pl.BlockSpec((din, d), lambda i: (0, 0)),
            pl.BlockSpec((din, d), lambda i: (0, 0)),
            pl.BlockSpec((d,), lambda i: (0,)),
        ],
        out_specs=[
            pl.BlockSpec((2, blk, d2), lambda i: (0, i, 0)),
            pl.BlockSpec((2, blk, d2), lambda i: (0, i, 0)),
        ],
        out_shape=[jax.ShapeDtypeStruct((2, n_pad, d2), jnp.float32)] * 2,
    )(x, wsrc, wdst, bmsg)
    return ta.reshape(2 * n_pad, d2), tb.reshape(2 * n_pad, d2)


def _tc_post(x, agg, wnx, wna0, wna1, bnode, wsk, bsk):
    n, din = x.shape
    dout = wnx.shape[1]
    d2 = agg.shape[2]
    blk = min(512, _round_up(n, 8))
    grid = (pl.cdiv(n, blk),)

    def body(x_ref, a0_ref, a1_ref, wnx_ref, wa0_ref, wa1_ref, bn_ref,
             wsk_ref, bsk_ref, o_ref):
        xb = x_ref[...]
        h = jnp.dot(xb, wnx_ref[...], preferred_element_type=jnp.float32)
        h = h + jnp.dot(a0_ref[0], wa0_ref[...], preferred_element_type=jnp.float32)
        h = h + jnp.dot(a1_ref[0], wa1_ref[...], preferred_element_type=jnp.float32)
        h = _leaky(h + bn_ref[...])
        o_ref[...] = h + jnp.dot(xb, wsk_ref[...],
                                 preferred_element_type=jnp.float32) + bsk_ref[...]

    return pl.pallas_call(
        body,
        grid=grid,
        in_specs=[
            pl.BlockSpec((blk, din), lambda i: (i, 0)),
            pl.BlockSpec((1, blk, d2), lambda i: (0, i, 0)),
            pl.BlockSpec((1, blk, d2), lambda i: (1, i, 0)),
            pl.BlockSpec((din, dout), lambda i: (0, 0)),
            pl.BlockSpec((d2, dout), lambda i: (0, 0)),
            pl.BlockSpec((d2, dout), lambda i: (0, 0)),
            pl.BlockSpec((dout,), lambda i: (0,)),
            pl.BlockSpec((din, dout), lambda i: (0, 0)),
            pl.BlockSpec((dout,), lambda i: (0,)),
        ],
        out_specs=pl.BlockSpec((blk, dout), lambda i: (i, 0)),
        out_shape=jax.ShapeDtypeStruct((n, dout), jnp.float32),
    )(x, agg, agg, wnx, wna0, wna1, bnode, wsk, bsk)


def _tc_bn(h, skip, gamma, beta):
    n, d = h.shape
    blk = 2560
    grid = (pl.cdiv(n, blk),)

    def sums_body(h_ref, s_ref, o_ref):
        y = h_ref[...] + s_ref[...]
        row = lax.broadcasted_iota(jnp.int32, (blk, 1), 0) + pl.program_id(0) * blk
        y = jnp.where(row < n, y, 0.0)

        @pl.when(pl.program_id(0) == 0)
        def _():
            o_ref[...] = jnp.zeros_like(o_ref)

        o_ref[0, :] += jnp.sum(y, axis=0)

    sums = pl.pallas_call(
        sums_body,
        grid=grid,
        in_specs=[pl.BlockSpec((blk, d), lambda i: (i, 0)),
                  pl.BlockSpec((blk, d), lambda i: (i, 0))],
        out_specs=pl.BlockSpec((8, d), lambda i: (0, 0)),
        out_shape=jax.ShapeDtypeStruct((8, d), jnp.float32),
    )(h, skip)

    def var_body(h_ref, s_ref, m_ref, o_ref):
        y = h_ref[...] + s_ref[...] - m_ref[0, :] / n
        row = lax.broadcasted_iota(jnp.int32, (blk, 1), 0) + pl.program_id(0) * blk
        y = jnp.where(row < n, y, 0.0)

        @pl.when(pl.program_id(0) == 0)
        def _():
            o_ref[...] = jnp.zeros_like(o_ref)

        o_ref[0, :] += jnp.sum(y * y, axis=0)

    vsums = pl.pallas_call(
        var_body,
        grid=grid,
        in_specs=[pl.BlockSpec((blk, d), lambda i: (i, 0)),
                  pl.BlockSpec((blk, d), lambda i: (i, 0)),
                  pl.BlockSpec((8, d), lambda i: (0, 0))],
        out_specs=pl.BlockSpec((8, d), lambda i: (0, 0)),
        out_shape=jax.ShapeDtypeStruct((8, d), jnp.float32),
    )(h, skip, sums)

    def norm_body(h_ref, s_ref, m_ref, v_ref, g_ref, b_ref, o_ref):
        y = h_ref[...] + s_ref[...]
        mu = m_ref[0, :] / n
        var = v_ref[0, :] / n
        o_ref[...] = _leaky((y - mu) / jnp.sqrt(var + 1e-5) * g_ref[...] + b_ref[...])

    return pl.pallas_call(
        norm_body,
        grid=grid,
        in_specs=[pl.BlockSpec((blk, d), lambda i: (i, 0)),
                  pl.BlockSpec((blk, d), lambda i: (i, 0)),
                  pl.BlockSpec((8, d), lambda i: (0, 0)),
                  pl.BlockSpec((8, d), lambda i: (0, 0)),
                  pl.BlockSpec((d,), lambda i: (0,)),
                  pl.BlockSpec((d,), lambda i: (0,))],
        out_specs=pl.BlockSpec((blk, d), lambda i: (i, 0)),
        out_shape=jax.ShapeDtypeStruct((n, d), jnp.float32),
    )(h, skip, sums, vsums, gamma, beta)


def _tc_dec(x, params):
    n = x.shape[0]
    blk = 2000
    grid = (pl.cdiv(n, blk),)

    def body(x_ref, w1_ref, b1_ref, w2_ref, b2_ref, g_ref, be_ref, o_ref):
        h = _leaky(jnp.dot(x_ref[...], w1_ref[...],
                           preferred_element_type=jnp.float32) + b1_ref[...])
        y = jnp.dot(h, w2_ref[...], preferred_element_type=jnp.float32) + b2_ref[...]
        mu = y.mean(axis=-1, keepdims=True)
        var = ((y - mu) ** 2).mean(axis=-1, keepdims=True)
        o_ref[...] = (y - mu) / jnp.sqrt(var + 1e-5) * g_ref[...] + be_ref[...]

    return pl.pallas_call(
        body,
        grid=grid,
        in_specs=[
            pl.BlockSpec((blk, 64), lambda i: (i, 0)),
            pl.BlockSpec((64, 32), lambda i: (0, 0)),
            pl.BlockSpec((32,), lambda i: (0,)),
            pl.BlockSpec((32, 3), lambda i: (0, 0)),
            pl.BlockSpec((3,), lambda i: (0,)),
            pl.BlockSpec((3,), lambda i: (0,)),
            pl.BlockSpec((3,), lambda i: (0,)),
        ],
        out_specs=pl.BlockSpec((blk, 3), lambda i: (i, 0)),
        out_shape=jax.ShapeDtypeStruct((n, 3), jnp.float32),
    )(x, params["dec1"]["W"], params["dec1"]["b"], params["dec2"]["W"],
      params["dec2"]["b"], params["ln"]["gamma"], params["ln"]["beta"])


def _chunk_c(n_pad, d2, t_per):
    budget = 8 * 1024 * 1024 - n_pad * d2 * 4 - 512 * 1024
    per_tile = budget // 16
    c_lim = max(128, per_tile // (16 * d2 + 32))
    best = 128
    for mult in range(1, c_lim // 128 + 1):
        c = 128 * mult
        if c <= c_lim and t_per % c == 0:
            best = c
    return best


def _edge_epad(e):
    return 2048 * ((e + 2047) // 2048)


@functools.lru_cache(maxsize=None)
def _edge_kernel(n_pad, d2, e_pad):
    t_per = e_pad // 16
    c = _chunk_c(n_pad, d2, t_per)
    n_chunks = t_per // c
    k = c // 128
    z_rows = n_pad // 16
    nvec = d2 // 16

    mesh = plsc.VectorSubcoreMesh(core_axis_name="c", subcore_axis_name="s")

    def body(ta, tb, esrc, edst, out, acc, ia, idr, ids, ibo, buf_a, buf_b,
             sem_g, sem_i, sem_s):
        cid = lax.axis_index("c")
        sid = lax.axis_index("s")
        tile_base = sid * t_per
        zbase = sid * z_rows
        off = cid * n_pad

        @pl.loop(0, c, unroll=8)
        def _zfill(r):
            for q in range(nvec):
                buf_a[0, r, pl.ds(q * 16, 16)] = jnp.zeros((16,), jnp.float32)

        zoff = 0
        while zoff < z_rows:
            sz = min(c, z_rows - zoff)
            pltpu.sync_copy(buf_a.at[0, pl.ds(0, sz)],
                            acc.at[pl.ds(zbase + zoff, sz)])
            zoff += sz
        plsc.subcore_barrier()

        def fire_idx(j, b):
            pltpu.async_copy(esrc.at[pl.ds(tile_base + j * c, c)],
                             ia.at[b], sem_i)
            pltpu.async_copy(edst.at[pl.ds(tile_base + j * c, c)],
                             idr.at[b], sem_i)

        def drain_idx_patch(b):
            pltpu.make_async_copy(esrc.at[pl.ds(0, c)], ia.at[b], sem_i).wait()
            pltpu.make_async_copy(esrc.at[pl.ds(0, c)], idr.at[b], sem_i).wait()

            @pl.loop(0, k)
            def _patch(j):
                for q in range(8):
                    sl = pl.ds(q * 16, 16)
                    p = j * 128 + q * 16
                    fl = pl.ds(p, 16)
                    ia[b, fl] = ia[b, fl] + off
                    raw = idr[b, fl]
                    ids[b, j, sl] = raw
                    ibo[b, fl] = raw + off

        def fire_gathers(b):
            for j in range(k):
                pltpu.async_copy(ta.at[ia.at[b, pl.ds(j * 128, 128)]],
                                 buf_a.at[b, pl.ds(j * 128, 128)], sem_g)
                pltpu.async_copy(tb.at[ibo.at[b, pl.ds(j * 128, 128)]],
                                 buf_b.at[b, pl.ds(j * 128, 128)], sem_g)

        def drain_gathers(b):
            for j in range(k):
                pltpu.make_async_copy(ta.at[ia.at[b, pl.ds(j * 128, 128)]],
                                      buf_a.at[b, pl.ds(j * 128, 128)],
                                      sem_g).wait()
                pltpu.make_async_copy(tb.at[ibo.at[b, pl.ds(j * 128, 128)]],
                                      buf_b.at[b, pl.ds(j * 128, 128)],
                                      sem_g).wait()

        def fire_scatter(b):
            for j in range(k):
                pltpu.async_copy(buf_a.at[b, pl.ds(j * 128, 128)],
                                 acc.at[ids.at[b, j]], sem_s, add=True)

        def drain_scatter(b):
            for j in range(k):
                pltpu.make_async_copy(buf_a.at[b, pl.ds(j * 128, 128)],
                                      acc.at[ids.at[b, j]], sem_s).wait()

        fire_idx(0, 0)
        if n_chunks > 1:
            fire_idx(1, 1)
        drain_idx_patch(0)
        fire_gathers(0)

        @pl.loop(0, n_chunks)
        def _chunks(j):
            b = jnp.bitwise_and(j, 1)
            nb = 1 - b
            drain_gathers(b)

            @pl.loop(0, c, unroll=4)
            def _comp(r):
                for q in range(nvec):
                    sl = pl.ds(q * 16, 16)
                    v = buf_a[b, r, sl] + buf_b[b, r, sl]
                    buf_a[b, r, sl] = jnp.maximum(v, LEAK * v)

            @pl.when(j >= 1)
            def _():
                drain_scatter(nb)

            fire_scatter(b)

            @pl.when(j + 1 < n_chunks)
            def _():
                drain_idx_patch(nb)
                fire_gathers(nb)

            @pl.when(j + 2 < n_chunks)
            def _():
                fire_idx(j + 2, b)

        drain_scatter((n_chunks - 1) & 1)

        plsc.subcore_barrier()
        zoff = 0
        while zoff < z_rows:
            sz = min(c, z_rows - zoff)
            pltpu.sync_copy(acc.at[pl.ds(zbase + zoff, sz)],
                            out.at[cid, pl.ds(zbase + zoff, sz)])
            zoff += sz

    return pl.kernel(
        body,
        out_type=jax.ShapeDtypeStruct((2, n_pad, d2), jnp.float32),
        mesh=mesh,
        compiler_params=pltpu.CompilerParams(use_tc_tiling_on_sc=False),
        scratch_types=[
            pltpu.VMEM_SHARED((n_pad, d2), jnp.float32),
            pltpu.VMEM((2, c), jnp.int32),
            pltpu.VMEM((2, c), jnp.int32),
            pltpu.VMEM((2, k, 128), jnp.int32),
            pltpu.VMEM((2, c), jnp.int32),
            pltpu.VMEM((2, c, d2), jnp.float32),
            pltpu.VMEM((2, c, d2), jnp.float32),
            pltpu.SemaphoreType.DMA,
            pltpu.SemaphoreType.DMA,
            pltpu.SemaphoreType.DMA,
        ],
    )


@functools.lru_cache(maxsize=None)
def _unpool_kernel(n_in, n_out_pad, d):
    rows = n_out_pad // 32
    cmax = max(128, min(1024, (256 * 1024) // (d * 4) // 8 * 8))
    mesh = plsc.VectorSubcoreMesh(core_axis_name="c", subcore_axis_name="s")

    def body(x_hbm, up_hbm, out, idx_v, buf, sem):
        cid = lax.axis_index("c")
        sid = lax.axis_index("s")
        wid = sid * 2 + cid
        base = wid * rows
        off = 0
        while off < rows:
            sz = min(cmax, rows - off)
            pltpu.sync_copy(up_hbm.at[pl.ds(base + off, sz)], idx_v.at[pl.ds(0, sz)])
            descs = []
            goff = 0
            while goff < sz:
                g = min(128, sz - goff)
                descs.append(pltpu.async_copy(
                    x_hbm.at[idx_v.at[pl.ds(goff, g)]],
                    buf.at[pl.ds(goff, g)], sem))
                goff += g
            for dsc in descs:
                dsc.wait()
            pltpu.sync_copy(buf.at[pl.ds(0, sz)], out.at[pl.ds(base + off, sz)])
            off += sz

    return pl.kernel(
        body,
        out_type=jax.ShapeDtypeStruct((n_out_pad, d), jnp.float32),
        mesh=mesh,
        compiler_params=pltpu.CompilerParams(use_tc_tiling_on_sc=False),
        scratch_types=[
            pltpu.VMEM((cmax,), jnp.int32),
            pltpu.VMEM((cmax, d), jnp.float32),
            pltpu.SemaphoreType.DMA,
        ],
    )


def _unpool(x, up, n_out):
    n_in, d = x.shape
    n_out_pad = _round_up(n_out, 256)
    up_pad = jnp.concatenate([up, jnp.zeros((n_out_pad - n_out,), jnp.int32)])
    out = _unpool_kernel(n_in, n_out_pad, d)(x, up_pad)
    return out[:n_out]


def _pad_edges(e, n):
    e_cnt = e.shape[1]
    e_pad = _edge_epad(e_cnt)
    fill = e_pad - e_cnt
    esrc = jnp.concatenate([e[0], jnp.zeros((fill,), jnp.int32)])
    edst = jnp.concatenate([e[1], jnp.full((fill,), n, jnp.int32)])
    return esrc, edst, e_pad


def _mpl(x, edges, mp):
    esrc, edst, e_pad = edges
    n, din = x.shape
    n_pad = _npad(n)
    d = mp["msg"]["W"].shape[1]
    d2 = d // 2
    ta, tb = _tc_pre(x, mp["msg"]["W"][:din], mp["msg"]["W"][din:],
                     mp["msg"]["b"], n_pad)
    agg = _edge_kernel(n_pad, d2, e_pad)(ta, tb, esrc, edst)
    wn = mp["node"]["W"]
    return _tc_post(x, agg, wn[:din], wn[din:din + d2], wn[din + d2:],
                    mp["node"]["b"], mp["skip"]["W"], mp["skip"]["b"])


N_BOT, N_MID, N_TOP = 3125, 12500, 50000


def kernel(z, e_bot, e_mid, e_top, up1, up2, params):
    p = params
    eb = _pad_edges(e_bot, N_BOT)
    em_mid = _pad_edges(e_mid, N_MID)
    et_top = _pad_edges(e_top, N_TOP)

    x0 = _tc_head(z, p)
    x1 = _mpl(x0, eb, p["mpl_bottom"])

    x1u = _unpool(x1, up1, N_MID)
    skip = _mpl(x1u, em_mid, p["l0_skip"])
    h = _mpl(x1, eb, p["l0_mpl1"])
    hu = _unpool(h, up1, N_MID)
    h = _mpl(hu, em_mid, p["l0_mpl2"])
    x2 = _tc_bn(h, skip, p["l0_bn"]["gamma"], p["l0_bn"]["beta"])

    x2u = _unpool(x2, up2, N_TOP)
    skip = _mpl(x2u, et_top, p["l1_skip"])
    h = _mpl(x2, em_mid, p["l1_mpl1"])
    hu = _unpool(h, up2, N_TOP)
    h = _mpl(hu, et_top, p["l1_mpl2"])
    x3 = _tc_bn(h, skip, p["l1_bn"]["gamma"], p["l1_bn"]["beta"])

    x4 = _mpl(x3, et_top, p["final"])
    return _tc_dec(x4, p)
